# baseline (device time: 14579 ns/iter reference)
import jax
import jax.numpy as jnp
from jax import lax
from jax.experimental import pallas as pl
from jax.experimental.pallas import tpu as pltpu

T = 256
V_LOC = 4096
NC = 4
VC = V_LOC // NC


def kernel(x, W, labels):
    labels2d = labels.reshape(T, 1)

    def body(x_hbm, w_hbm, lbl_hbm, out_ref,
             xv, wv, lblv, comm_ref,
             ldma_sems, wdma_sems, send_sem, recv_sem):
        my_x = lax.axis_index("x")
        my_y = lax.axis_index("y")
        peer = (my_x, 1 - my_y)

        barrier_sem = pltpu.get_barrier_semaphore()
        pl.semaphore_signal(
            barrier_sem, inc=1, device_id=peer,
            device_id_type=pl.DeviceIdType.MESH,
        )

        x_dma = pltpu.make_async_copy(x_hbm, xv, ldma_sems.at[0])
        x_dma.start()
        lbl_dma = pltpu.make_async_copy(lbl_hbm, lblv, ldma_sems.at[1])
        lbl_dma.start()
        w_dmas = [
            pltpu.make_async_copy(
                w_hbm.at[:, pl.ds(c * VC, VC)], wv.at[c], wdma_sems.at[c]
            )
            for c in range(NC)
        ]
        for d in w_dmas:
            d.start()

        x_dma.wait()
        lbl_dma.wait()
        xb = xv[...].astype(jnp.bfloat16)
        lbl = lblv[...] - my_y * V_LOC

        s = jnp.zeros((T, 1), jnp.float32)
        lab = jnp.zeros((T, 1), jnp.float32)
        for c in range(NC):
            w_dmas[c].wait()
            logits = jnp.dot(
                xb, wv[c].astype(jnp.bfloat16),
                preferred_element_type=jnp.float32,
            )
            s = s + jnp.sum(jnp.exp(logits), axis=1, keepdims=True)
            cols = c * VC + lax.broadcasted_iota(jnp.int32, (T, VC), 1)
            lab = lab + jnp.sum(
                jnp.where(cols == lbl, logits, 0.0), axis=1, keepdims=True
            )

        comm_ref[0, :, :] = jnp.concatenate([s, lab], axis=1).T
        pl.semaphore_wait(barrier_sem, 1)

        rdma = pltpu.make_async_remote_copy(
            src_ref=comm_ref.at[0],
            dst_ref=comm_ref.at[1],
            send_sem=send_sem,
            recv_sem=recv_sem,
            device_id=peer,
            device_id_type=pl.DeviceIdType.MESH,
        )
        rdma.start()
        rdma.wait()

        tot = comm_ref[0, :, :] + comm_ref[1, :, :]
        out_ref[...] = jnp.log(tot[0]) - tot[1]

    return pl.pallas_call(
        body,
        out_shape=jax.ShapeDtypeStruct((T,), jnp.float32),
        in_specs=[
            pl.BlockSpec(memory_space=pl.ANY),
            pl.BlockSpec(memory_space=pl.ANY),
            pl.BlockSpec(memory_space=pl.ANY),
        ],
        out_specs=pl.BlockSpec(memory_space=pltpu.VMEM),
        scratch_shapes=[
            pltpu.VMEM((T, 512), jnp.float32),
            pltpu.VMEM((NC, 512, VC), jnp.float32),
            pltpu.VMEM((T, 1), jnp.int32),
            pltpu.VMEM((2, 2, T), jnp.float32),
            pltpu.SemaphoreType.DMA((2,)),
            pltpu.SemaphoreType.DMA((NC,)),
            pltpu.SemaphoreType.DMA,
            pltpu.SemaphoreType.DMA,
        ],
        compiler_params=pltpu.CompilerParams(collective_id=0),
    )(x, W, labels2d)
